# baseline (speedup 1.0000x reference)
import functools

import jax
import jax.numpy as jnp
from jax import lax
from jax.experimental import pallas as pl
from jax.experimental.pallas import tpu as pltpu
from jax.experimental.pallas import tpu_sc as plsc

N_NODES = 10000
N_EDGES = 320000
D_IN = 128
D_OUT = 16

NC = 2
NS = 16
NW = NC * NS
EPW = N_EDGES // NW
CHUNK = 80
NCHUNK = 128
PHCH = 64
EPW_PAD = NCHUNK * CHUNK
N_PAD = N_NODES + 16
ROWS_MAIN = 632
ROWS_LAST = N_NODES - (NS - 1) * ROWS_MAIN


def _seg_sum_partials(src2d, dst2d, x, d):
    mesh = plsc.VectorSubcoreMesh(core_axis_name="c", subcore_axis_name="s")

    @functools.partial(
        pl.kernel,
        out_type=jax.ShapeDtypeStruct((NC, N_NODES, d), jnp.float32),
        mesh=mesh,
        scratch_types=[
            pltpu.VMEM((NCHUNK, CHUNK), jnp.int32),
            pltpu.VMEM((NCHUNK, CHUNK), jnp.int32),
            pltpu.VMEM((CHUNK, d), jnp.float32),
            pltpu.VMEM((CHUNK, d), jnp.float32),
            pltpu.VMEM((8, d), jnp.float32),
            pltpu.VMEM_SHARED((N_PAD, d), jnp.float32),
            pltpu.SemaphoreType.DMA,
            pltpu.SemaphoreType.DMA,
        ],
    )
    def body(src_hbm, dst_hbm, x_hbm, out_hbm, sidx, didx, rows_a, rows_b,
             zbuf, acc, sem_a, sem_b):
        cid = lax.axis_index("c")
        sid = lax.axis_index("s")
        wid = sid * NC + cid

        for i in range(8):
            for j in range(d // 16):
                zbuf[i, pl.ds(j * 16, 16)] = jnp.zeros((16,), jnp.float32)
        base = sid * ROWS_MAIN
        nrows = jnp.where(sid == NS - 1, ROWS_LAST, ROWS_MAIN)

        def zcopy(k, carry):
            pltpu.sync_copy(zbuf, acc.at[pl.ds(base + k * 8, 8)])
            return carry

        lax.fori_loop(0, nrows // 8, zcopy, 0)
        plsc.subcore_barrier()

        pltpu.sync_copy(src_hbm.at[wid], sidx)
        pltpu.sync_copy(dst_hbm.at[wid], didx)

        def step(i, carry):
            pltpu.async_copy(x_hbm.at[sidx.at[i]], rows_a, sem_a).wait()
            pltpu.sync_copy(rows_a, acc.at[didx.at[i]], add=True)
            return carry

        lax.fori_loop(0, NCHUNK, step, 0)

        plsc.subcore_barrier()

        @pl.when(sid < NS - 1)
        def _():
            pltpu.sync_copy(acc.at[pl.ds(base, ROWS_MAIN)],
                            out_hbm.at[cid, pl.ds(base, ROWS_MAIN)])

        @pl.when(sid == NS - 1)
        def _():
            pltpu.sync_copy(acc.at[pl.ds(base, ROWS_LAST)],
                            out_hbm.at[cid, pl.ds(base, ROWS_LAST)])

    return body(src2d, dst2d, x)


def _dense_mid(parts, W1, b1):
    R = 400

    def body(a_ref, w1_ref, b1_ref, o_ref):
        a = a_ref[0] + a_ref[1]
        h = jnp.dot(a, w1_ref[...], preferred_element_type=jnp.float32)
        o_ref[...] = jnp.maximum(h + b1_ref[...], 0.0)

    return pl.pallas_call(
        body,
        grid=(N_NODES // R,),
        in_specs=[
            pl.BlockSpec((NC, R, D_IN), lambda i: (0, i, 0)),
            pl.BlockSpec((D_IN, D_IN), lambda i: (0, 0)),
            pl.BlockSpec((1, D_IN), lambda i: (0, 0)),
        ],
        out_specs=pl.BlockSpec((R, D_IN), lambda i: (i, 0)),
        out_shape=jax.ShapeDtypeStruct((N_NODES, D_IN), jnp.float32),
    )(parts, W1, b1.reshape(1, D_IN))


def _final_logsoftmax(parts2, W2, b2):
    R = 400

    def body(a_ref, w2_ref, b2_ref, o_ref):
        a = a_ref[0] + a_ref[1]
        v = jnp.dot(a, w2_ref[...], preferred_element_type=jnp.float32)
        v = v + b2_ref[...]
        m = jnp.max(v, axis=1, keepdims=True)
        e = jnp.exp(v - m)
        s = jnp.sum(e, axis=1, keepdims=True)
        o_ref[...] = (v - m) - jnp.log(s)

    return pl.pallas_call(
        body,
        grid=(N_NODES // R,),
        in_specs=[
            pl.BlockSpec((NC, R, D_IN), lambda i: (0, i, 0)),
            pl.BlockSpec((D_IN, D_OUT), lambda i: (0, 0)),
            pl.BlockSpec((1, D_OUT), lambda i: (0, 0)),
        ],
        out_specs=pl.BlockSpec((R, D_OUT), lambda i: (i, 0)),
        out_shape=jax.ShapeDtypeStruct((N_NODES, D_OUT), jnp.float32),
    )(parts2, W2, b2.reshape(1, D_OUT))


def kernel(x, edge_index, W1, b1, W2, b2):
    pad = EPW_PAD - EPW
    src = edge_index[0].astype(jnp.int32).reshape(NW, EPW)
    src = jnp.pad(src, ((0, 0), (0, pad))).reshape(NW, NCHUNK, CHUNK)
    dst = edge_index[1].astype(jnp.int32).reshape(NW, EPW)
    pad_rows = N_NODES + (jnp.arange(NW, dtype=jnp.int32) // NC)
    dst = jnp.concatenate(
        [dst, jnp.broadcast_to(pad_rows[:, None], (NW, pad))], axis=1)
    dst = dst.reshape(NW, NCHUNK, CHUNK)
    parts1 = _seg_sum_partials(src, dst, x, D_IN)
    h = _dense_mid(parts1, W1, b1)
    parts2 = _seg_sum_partials(src, dst, h, D_IN)
    return _final_logsoftmax(parts2, W2, b2)


# speedup vs baseline: 2.6847x; 2.6847x over previous
import functools

import jax
import jax.numpy as jnp
from jax import lax
from jax.experimental import pallas as pl
from jax.experimental.pallas import tpu as pltpu
from jax.experimental.pallas import tpu_sc as plsc

N_NODES = 10000
N_EDGES = 320000
D_IN = 128
D_OUT = 16

NC = 2
NS = 16
NW = NC * NS
EPW = N_EDGES // NW
CHUNK = 80
NCHUNK = EPW // CHUNK
PHASES = ((0, 64), (64, NCHUNK - 64))
PHBUF = 64
ROWS_MAIN = 632
ROWS_LAST = N_NODES - (NS - 1) * ROWS_MAIN


def _seg_sum_partials(src2d, dst2d, x, d):
    mesh = plsc.VectorSubcoreMesh(core_axis_name="c", subcore_axis_name="s")

    @functools.partial(
        pl.kernel,
        out_type=jax.ShapeDtypeStruct((NC, N_NODES, d), jnp.float32),
        mesh=mesh,
        scratch_types=[
            pltpu.VMEM((PHBUF, CHUNK), jnp.int32),
            pltpu.VMEM((PHBUF, CHUNK), jnp.int32),
            pltpu.VMEM((CHUNK, d), jnp.float32),
            pltpu.VMEM((CHUNK, d), jnp.float32),
            pltpu.VMEM((8, d), jnp.float32),
            pltpu.VMEM_SHARED((N_NODES, d), jnp.float32),
            pltpu.SemaphoreType.DMA,
            pltpu.SemaphoreType.DMA,
        ],
    )
    def body(src_hbm, dst_hbm, x_hbm, out_hbm, sidx, didx, rows_a, rows_b,
             zbuf, acc, sem_a, sem_b):
        cid = lax.axis_index("c")
        sid = lax.axis_index("s")
        wid = sid * NC + cid

        for i in range(8):
            for j in range(d // 16):
                zbuf[i, pl.ds(j * 16, 16)] = jnp.zeros((16,), jnp.float32)
        base = sid * ROWS_MAIN
        nrows = jnp.where(sid == NS - 1, ROWS_LAST, ROWS_MAIN)

        def zcopy(k, carry):
            pltpu.sync_copy(zbuf, acc.at[pl.ds(base + k * 8, 8)])
            return carry

        lax.fori_loop(0, nrows // 8, zcopy, 0)
        plsc.subcore_barrier()

        for start, cnt in PHASES:
            pltpu.sync_copy(src_hbm.at[wid, pl.ds(start, cnt)],
                            sidx.at[pl.ds(0, cnt)])
            pltpu.sync_copy(dst_hbm.at[wid, pl.ds(start, cnt)],
                            didx.at[pl.ds(0, cnt)])
            pltpu.async_copy(x_hbm.at[sidx.at[0]], rows_a, sem_a)

            def pair(k, carry):
                i = 2 * k
                pltpu.make_async_copy(x_hbm.at[sidx.at[i]], rows_a, sem_a).wait()
                pltpu.async_copy(x_hbm.at[sidx.at[i + 1]], rows_b, sem_b)
                pltpu.sync_copy(rows_a, acc.at[didx.at[i]], add=True)
                pltpu.make_async_copy(x_hbm.at[sidx.at[i + 1]], rows_b,
                                      sem_b).wait()

                @pl.when(i + 2 < cnt)
                def _():
                    pltpu.async_copy(x_hbm.at[sidx.at[i + 2]], rows_a, sem_a)

                pltpu.sync_copy(rows_b, acc.at[didx.at[i + 1]], add=True)
                return carry

            lax.fori_loop(0, cnt // 2, pair, 0)
            if cnt % 2:
                pltpu.make_async_copy(x_hbm.at[sidx.at[cnt - 1]], rows_a,
                                      sem_a).wait()
                pltpu.sync_copy(rows_a, acc.at[didx.at[cnt - 1]], add=True)

        plsc.subcore_barrier()

        @pl.when(sid < NS - 1)
        def _():
            pltpu.sync_copy(acc.at[pl.ds(base, ROWS_MAIN)],
                            out_hbm.at[cid, pl.ds(base, ROWS_MAIN)])

        @pl.when(sid == NS - 1)
        def _():
            pltpu.sync_copy(acc.at[pl.ds(base, ROWS_LAST)],
                            out_hbm.at[cid, pl.ds(base, ROWS_LAST)])

    return body(src2d, dst2d, x)


def _dense_mid(parts, W1, b1):
    R = 400

    def body(a_ref, w1_ref, b1_ref, o_ref):
        a = a_ref[0] + a_ref[1]
        h = jnp.dot(a, w1_ref[...], preferred_element_type=jnp.float32)
        o_ref[...] = jnp.maximum(h + b1_ref[...], 0.0)

    return pl.pallas_call(
        body,
        grid=(N_NODES // R,),
        in_specs=[
            pl.BlockSpec((NC, R, D_IN), lambda i: (0, i, 0)),
            pl.BlockSpec((D_IN, D_IN), lambda i: (0, 0)),
            pl.BlockSpec((1, D_IN), lambda i: (0, 0)),
        ],
        out_specs=pl.BlockSpec((R, D_IN), lambda i: (i, 0)),
        out_shape=jax.ShapeDtypeStruct((N_NODES, D_IN), jnp.float32),
    )(parts, W1, b1.reshape(1, D_IN))


def _final_logsoftmax(parts2, W2, b2):
    R = 400

    def body(a_ref, w2_ref, b2_ref, o_ref):
        a = a_ref[0] + a_ref[1]
        v = jnp.dot(a, w2_ref[...], preferred_element_type=jnp.float32)
        v = v + b2_ref[...]
        m = jnp.max(v, axis=1, keepdims=True)
        e = jnp.exp(v - m)
        s = jnp.sum(e, axis=1, keepdims=True)
        o_ref[...] = (v - m) - jnp.log(s)

    return pl.pallas_call(
        body,
        grid=(N_NODES // R,),
        in_specs=[
            pl.BlockSpec((NC, R, D_IN), lambda i: (0, i, 0)),
            pl.BlockSpec((D_IN, D_OUT), lambda i: (0, 0)),
            pl.BlockSpec((1, D_OUT), lambda i: (0, 0)),
        ],
        out_specs=pl.BlockSpec((R, D_OUT), lambda i: (i, 0)),
        out_shape=jax.ShapeDtypeStruct((N_NODES, D_OUT), jnp.float32),
    )(parts2, W2, b2.reshape(1, D_OUT))


def kernel(x, edge_index, W1, b1, W2, b2):
    src = edge_index[0].astype(jnp.int32).reshape(NW, NCHUNK, CHUNK)
    dst = edge_index[1].astype(jnp.int32).reshape(NW, NCHUNK, CHUNK)
    parts1 = _seg_sum_partials(src, dst, x, D_IN)
    h = _dense_mid(parts1, W1, b1)
    parts2 = _seg_sum_partials(src, dst, h, D_IN)
    return _final_logsoftmax(parts2, W2, b2)


# speedup vs baseline: 3.7361x; 1.3916x over previous
import functools

import jax
import jax.numpy as jnp
from jax import lax
from jax.experimental import pallas as pl
from jax.experimental.pallas import tpu as pltpu
from jax.experimental.pallas import tpu_sc as plsc

N_NODES = 10000
N_EDGES = 320000
D_IN = 128
D_OUT = 16

NC = 2
NS = 16
NW = NC * NS
EPW = N_EDGES // NW
CHUNK = 80
NCHUNK = EPW // CHUNK
PHASES = ((0, 64), (64, NCHUNK - 64))
PHBUF = 64
ROWS_MAIN = 632
ROWS_LAST = N_NODES - (NS - 1) * ROWS_MAIN


def _seg_sum_partials(src2d, dst2d, x, d):
    mesh = plsc.VectorSubcoreMesh(core_axis_name="c", subcore_axis_name="s")

    @functools.partial(
        pl.kernel,
        out_type=jax.ShapeDtypeStruct((NC, N_NODES, d), jnp.float32),
        mesh=mesh,
        scratch_types=[
            pltpu.VMEM((PHBUF, CHUNK), jnp.int32),
            pltpu.VMEM((PHBUF, CHUNK), jnp.int32),
            pltpu.VMEM((CHUNK, d), jnp.float32),
            pltpu.VMEM((CHUNK, d), jnp.float32),
            pltpu.VMEM((CHUNK, d), jnp.float32),
            pltpu.VMEM((8, d), jnp.float32),
            pltpu.VMEM_SHARED((N_NODES, d), jnp.float32),
            pltpu.SemaphoreType.DMA,
            pltpu.SemaphoreType.DMA,
            pltpu.SemaphoreType.DMA,
            pltpu.SemaphoreType.DMA,
            pltpu.SemaphoreType.DMA,
            pltpu.SemaphoreType.DMA,
        ],
    )
    def body(src_hbm, dst_hbm, x_hbm, out_hbm, sidx, didx, rows_0, rows_1,
             rows_2, zbuf, acc, gs_0, gs_1, gs_2, ss_0, ss_1, ss_2):
        rows = (rows_0, rows_1, rows_2)
        gsem = (gs_0, gs_1, gs_2)
        ssem = (ss_0, ss_1, ss_2)
        cid = lax.axis_index("c")
        sid = lax.axis_index("s")
        wid = sid * NC + cid

        for i in range(8):
            for j in range(d // 16):
                zbuf[i, pl.ds(j * 16, 16)] = jnp.zeros((16,), jnp.float32)
        base = sid * ROWS_MAIN
        nrows = jnp.where(sid == NS - 1, ROWS_LAST, ROWS_MAIN)

        def zcopy(k, carry):
            pltpu.sync_copy(zbuf, acc.at[pl.ds(base + k * 8, 8)])
            return carry

        lax.fori_loop(0, nrows // 8, zcopy, 0)
        plsc.subcore_barrier()

        for start, cnt in PHASES:
            pltpu.sync_copy(src_hbm.at[wid, pl.ds(start, cnt)],
                            sidx.at[pl.ds(0, cnt)])
            pltpu.sync_copy(dst_hbm.at[wid, pl.ds(start, cnt)],
                            didx.at[pl.ds(0, cnt)])
            pltpu.async_copy(x_hbm.at[sidx.at[0]], rows[0], gsem[0])
            pltpu.async_copy(x_hbm.at[sidx.at[1]], rows[1], gsem[1])

            def step(i, carry):
                p = lax.rem(i, 3)
                for b in range(3):
                    nb = (b + 2) % 3

                    @pl.when(p == b)
                    def _():
                        pltpu.make_async_copy(x_hbm.at[sidx.at[i]], rows[b],
                                              gsem[b]).wait()
                        pltpu.async_copy(rows[b], acc.at[didx.at[i]], ssem[b],
                                         add=True)

                        @pl.when(i + 2 < cnt)
                        def _():
                            @pl.when(i >= 1)
                            def _():
                                pltpu.make_async_copy(
                                    rows[nb], acc.at[didx.at[i]],
                                    ssem[nb]).wait()

                            pltpu.async_copy(x_hbm.at[sidx.at[i + 2]],
                                             rows[nb], gsem[nb])

                return carry

            lax.fori_loop(0, cnt, step, 0)
            for b in range(3):
                pltpu.make_async_copy(rows[b], acc.at[didx.at[0]],
                                      ssem[b]).wait()

        plsc.subcore_barrier()

        @pl.when(sid < NS - 1)
        def _():
            pltpu.sync_copy(acc.at[pl.ds(base, ROWS_MAIN)],
                            out_hbm.at[cid, pl.ds(base, ROWS_MAIN)])

        @pl.when(sid == NS - 1)
        def _():
            pltpu.sync_copy(acc.at[pl.ds(base, ROWS_LAST)],
                            out_hbm.at[cid, pl.ds(base, ROWS_LAST)])

    return body(src2d, dst2d, x)


def _dense_mid(parts, W1, b1):
    R = 400

    def body(a_ref, w1_ref, b1_ref, o_ref):
        a = a_ref[0] + a_ref[1]
        h = jnp.dot(a, w1_ref[...], preferred_element_type=jnp.float32)
        o_ref[...] = jnp.maximum(h + b1_ref[...], 0.0)

    return pl.pallas_call(
        body,
        grid=(N_NODES // R,),
        in_specs=[
            pl.BlockSpec((NC, R, D_IN), lambda i: (0, i, 0)),
            pl.BlockSpec((D_IN, D_IN), lambda i: (0, 0)),
            pl.BlockSpec((1, D_IN), lambda i: (0, 0)),
        ],
        out_specs=pl.BlockSpec((R, D_IN), lambda i: (i, 0)),
        out_shape=jax.ShapeDtypeStruct((N_NODES, D_IN), jnp.float32),
    )(parts, W1, b1.reshape(1, D_IN))


def _final_logsoftmax(parts2, W2, b2):
    R = 400

    def body(a_ref, w2_ref, b2_ref, o_ref):
        a = a_ref[0] + a_ref[1]
        v = jnp.dot(a, w2_ref[...], preferred_element_type=jnp.float32)
        v = v + b2_ref[...]
        m = jnp.max(v, axis=1, keepdims=True)
        e = jnp.exp(v - m)
        s = jnp.sum(e, axis=1, keepdims=True)
        o_ref[...] = (v - m) - jnp.log(s)

    return pl.pallas_call(
        body,
        grid=(N_NODES // R,),
        in_specs=[
            pl.BlockSpec((NC, R, D_IN), lambda i: (0, i, 0)),
            pl.BlockSpec((D_IN, D_OUT), lambda i: (0, 0)),
            pl.BlockSpec((1, D_OUT), lambda i: (0, 0)),
        ],
        out_specs=pl.BlockSpec((R, D_OUT), lambda i: (i, 0)),
        out_shape=jax.ShapeDtypeStruct((N_NODES, D_OUT), jnp.float32),
    )(parts2, W2, b2.reshape(1, D_OUT))


def kernel(x, edge_index, W1, b1, W2, b2):
    src = edge_index[0].astype(jnp.int32).reshape(NW, NCHUNK, CHUNK)
    dst = edge_index[1].astype(jnp.int32).reshape(NW, NCHUNK, CHUNK)
    parts1 = _seg_sum_partials(src, dst, x, D_IN)
    h = _dense_mid(parts1, W1, b1)
    parts2 = _seg_sum_partials(src, dst, h, D_IN)
    return _final_logsoftmax(parts2, W2, b2)


# speedup vs baseline: 3.8437x; 1.0288x over previous
import functools

import jax
import jax.numpy as jnp
from jax import lax
from jax.experimental import pallas as pl
from jax.experimental.pallas import tpu as pltpu
from jax.experimental.pallas import tpu_sc as plsc

N_NODES = 10000
N_EDGES = 320000
D_IN = 128
D_OUT = 16

NC = 2
NS = 16
NW = NC * NS
EPW = N_EDGES // NW
CHUNK = 80
NCHUNK = EPW // CHUNK
PHASES = ((0, 64), (64, NCHUNK - 64))
PHBUF = 64
ROWS_MAIN = 632
ROWS_LAST = N_NODES - (NS - 1) * ROWS_MAIN


def _seg_sum_partials(src2d, dst2d, x, d):
    mesh = plsc.VectorSubcoreMesh(core_axis_name="c", subcore_axis_name="s")

    @functools.partial(
        pl.kernel,
        out_type=jax.ShapeDtypeStruct((NC, N_NODES, d), jnp.float32),
        mesh=mesh,
        scratch_types=[
            pltpu.VMEM((PHBUF, CHUNK), jnp.int32),
            pltpu.VMEM((PHBUF, CHUNK), jnp.int32),
            pltpu.VMEM((CHUNK, d), jnp.float32),
            pltpu.VMEM((CHUNK, d), jnp.float32),
            pltpu.VMEM((CHUNK, d), jnp.float32),
            pltpu.VMEM_SHARED((N_NODES, d), jnp.float32),
            pltpu.SemaphoreType.DMA,
            pltpu.SemaphoreType.DMA,
            pltpu.SemaphoreType.DMA,
            pltpu.SemaphoreType.DMA,
            pltpu.SemaphoreType.DMA,
            pltpu.SemaphoreType.DMA,
        ],
    )
    def body(src_hbm, dst_hbm, x_hbm, out_hbm, sidx, didx, rows_0, rows_1,
             rows_2, acc, gs_0, gs_1, gs_2, ss_0, ss_1, ss_2):
        rows = (rows_0, rows_1, rows_2)
        gsem = (gs_0, gs_1, gs_2)
        ssem = (ss_0, ss_1, ss_2)
        cid = lax.axis_index("c")
        sid = lax.axis_index("s")
        wid = sid * NC + cid

        def zrow(i, carry):
            for j in range(d // 16):
                rows_0[i, pl.ds(j * 16, 16)] = jnp.zeros((16,), jnp.float32)
            return carry

        lax.fori_loop(0, CHUNK, zrow, 0)
        base = sid * ROWS_MAIN

        def zfill(nfull, tail):
            for k in range(nfull):
                pltpu.async_copy(rows_0, acc.at[pl.ds(base + k * CHUNK, CHUNK)],
                                 gs_0)
            pltpu.async_copy(rows_0.at[pl.ds(0, tail)],
                             acc.at[pl.ds(base + nfull * CHUNK, tail)], gs_0)
            for k in range(nfull):
                pltpu.make_async_copy(rows_0, acc.at[pl.ds(base, CHUNK)],
                                      gs_0).wait()
            pltpu.make_async_copy(rows_0.at[pl.ds(0, tail)],
                                  acc.at[pl.ds(base, tail)], gs_0).wait()

        @pl.when(sid < NS - 1)
        def _():
            zfill(ROWS_MAIN // CHUNK, ROWS_MAIN % CHUNK)

        @pl.when(sid == NS - 1)
        def _():
            zfill(ROWS_LAST // CHUNK, ROWS_LAST % CHUNK)

        plsc.subcore_barrier()

        for start, cnt in PHASES:
            pltpu.sync_copy(src_hbm.at[wid, pl.ds(start, cnt)],
                            sidx.at[pl.ds(0, cnt)])
            pltpu.sync_copy(dst_hbm.at[wid, pl.ds(start, cnt)],
                            didx.at[pl.ds(0, cnt)])
            pltpu.async_copy(x_hbm.at[sidx.at[0]], rows[0], gsem[0])
            pltpu.async_copy(x_hbm.at[sidx.at[1]], rows[1], gsem[1])

            def step(i, carry):
                p = lax.rem(i, 3)
                for b in range(3):
                    nb = (b + 2) % 3

                    @pl.when(p == b)
                    def _():
                        pltpu.make_async_copy(x_hbm.at[sidx.at[i]], rows[b],
                                              gsem[b]).wait()
                        pltpu.async_copy(rows[b], acc.at[didx.at[i]], ssem[b],
                                         add=True)

                        @pl.when(i + 2 < cnt)
                        def _():
                            @pl.when(i >= 1)
                            def _():
                                pltpu.make_async_copy(
                                    rows[nb], acc.at[didx.at[i]],
                                    ssem[nb]).wait()

                            pltpu.async_copy(x_hbm.at[sidx.at[i + 2]],
                                             rows[nb], gsem[nb])

                return carry

            lax.fori_loop(0, cnt, step, 0)
            for b in range(3):
                pltpu.make_async_copy(rows[b], acc.at[didx.at[0]],
                                      ssem[b]).wait()

        plsc.subcore_barrier()

        @pl.when(sid < NS - 1)
        def _():
            pltpu.sync_copy(acc.at[pl.ds(base, ROWS_MAIN)],
                            out_hbm.at[cid, pl.ds(base, ROWS_MAIN)])

        @pl.when(sid == NS - 1)
        def _():
            pltpu.sync_copy(acc.at[pl.ds(base, ROWS_LAST)],
                            out_hbm.at[cid, pl.ds(base, ROWS_LAST)])

    return body(src2d, dst2d, x)


def _dense_mid(parts, W1, b1):
    R = 400

    def body(a_ref, w1_ref, b1_ref, o_ref):
        a = a_ref[0] + a_ref[1]
        h = jnp.dot(a, w1_ref[...], preferred_element_type=jnp.float32)
        o_ref[...] = jnp.maximum(h + b1_ref[...], 0.0)

    return pl.pallas_call(
        body,
        grid=(N_NODES // R,),
        in_specs=[
            pl.BlockSpec((NC, R, D_IN), lambda i: (0, i, 0)),
            pl.BlockSpec((D_IN, D_IN), lambda i: (0, 0)),
            pl.BlockSpec((1, D_IN), lambda i: (0, 0)),
        ],
        out_specs=pl.BlockSpec((R, D_IN), lambda i: (i, 0)),
        out_shape=jax.ShapeDtypeStruct((N_NODES, D_IN), jnp.float32),
    )(parts, W1, b1.reshape(1, D_IN))


def _final_logsoftmax(parts2, W2, b2):
    R = 400

    def body(a_ref, w2_ref, b2_ref, o_ref):
        a = a_ref[0] + a_ref[1]
        v = jnp.dot(a, w2_ref[...], preferred_element_type=jnp.float32)
        v = v + b2_ref[...]
        m = jnp.max(v, axis=1, keepdims=True)
        e = jnp.exp(v - m)
        s = jnp.sum(e, axis=1, keepdims=True)
        o_ref[...] = (v - m) - jnp.log(s)

    return pl.pallas_call(
        body,
        grid=(N_NODES // R,),
        in_specs=[
            pl.BlockSpec((NC, R, D_IN), lambda i: (0, i, 0)),
            pl.BlockSpec((D_IN, D_OUT), lambda i: (0, 0)),
            pl.BlockSpec((1, D_OUT), lambda i: (0, 0)),
        ],
        out_specs=pl.BlockSpec((R, D_OUT), lambda i: (i, 0)),
        out_shape=jax.ShapeDtypeStruct((N_NODES, D_OUT), jnp.float32),
    )(parts2, W2, b2.reshape(1, D_OUT))


def kernel(x, edge_index, W1, b1, W2, b2):
    src = edge_index[0].astype(jnp.int32).reshape(NW, NCHUNK, CHUNK)
    dst = edge_index[1].astype(jnp.int32).reshape(NW, NCHUNK, CHUNK)
    parts1 = _seg_sum_partials(src, dst, x, D_IN)
    h = _dense_mid(parts1, W1, b1)
    parts2 = _seg_sum_partials(src, dst, h, D_IN)
    return _final_logsoftmax(parts2, W2, b2)


# speedup vs baseline: 3.8903x; 1.0121x over previous
import functools

import jax
import jax.numpy as jnp
from jax import lax
from jax.experimental import pallas as pl
from jax.experimental.pallas import tpu as pltpu
from jax.experimental.pallas import tpu_sc as plsc

N_NODES = 10000
N_EDGES = 320000
D_IN = 128
D_OUT = 16

NC = 2
NS = 16
NW = NC * NS
EPW = N_EDGES // NW
CHUNK = 80
NCHUNK = EPW // CHUNK
BLOCKS = ((0, 32), (32, 32), (64, 32), (96, NCHUNK - 96))
IBUF = 32
ROWS_MAIN = 632
ROWS_LAST = N_NODES - (NS - 1) * ROWS_MAIN


def _seg_sum_partials(src2d, dst2d, x, d):
    mesh = plsc.VectorSubcoreMesh(core_axis_name="c", subcore_axis_name="s")

    @functools.partial(
        pl.kernel,
        out_type=jax.ShapeDtypeStruct((NC, N_NODES, d), jnp.float32),
        mesh=mesh,
        scratch_types=[
            pltpu.VMEM((IBUF, CHUNK), jnp.int32),
            pltpu.VMEM((IBUF, CHUNK), jnp.int32),
            pltpu.VMEM((IBUF, CHUNK), jnp.int32),
            pltpu.VMEM((IBUF, CHUNK), jnp.int32),
            pltpu.VMEM((CHUNK, d), jnp.float32),
            pltpu.VMEM((CHUNK, d), jnp.float32),
            pltpu.VMEM((CHUNK, d), jnp.float32),
            pltpu.VMEM_SHARED((N_NODES, d), jnp.float32),
            pltpu.SemaphoreType.DMA,
            pltpu.SemaphoreType.DMA,
            pltpu.SemaphoreType.DMA,
            pltpu.SemaphoreType.DMA,
            pltpu.SemaphoreType.DMA,
            pltpu.SemaphoreType.DMA,
            pltpu.SemaphoreType.DMA,
        ],
    )
    def body(src_hbm, dst_hbm, x_hbm, out_hbm, sidx_a, didx_a, sidx_b, didx_b,
             rows_0, rows_1, rows_2, acc, gs_0, gs_1, gs_2, ss_0, ss_1, ss_2,
             isem):
        rows = (rows_0, rows_1, rows_2)
        gsem = (gs_0, gs_1, gs_2)
        ssem = (ss_0, ss_1, ss_2)
        sets = ((sidx_a, didx_a), (sidx_b, didx_b))
        cid = lax.axis_index("c")
        sid = lax.axis_index("s")
        wid = sid * NC + cid

        def zrow(i, carry):
            for j in range(d // 16):
                rows_0[i, pl.ds(j * 16, 16)] = jnp.zeros((16,), jnp.float32)
            return carry

        lax.fori_loop(0, CHUNK, zrow, 0)
        base = sid * ROWS_MAIN

        def zfill(nfull, tail):
            for k in range(nfull):
                pltpu.async_copy(rows_0, acc.at[pl.ds(base + k * CHUNK, CHUNK)],
                                 gs_0)
            pltpu.async_copy(rows_0.at[pl.ds(0, tail)],
                             acc.at[pl.ds(base + nfull * CHUNK, tail)], gs_0)
            for k in range(nfull):
                pltpu.make_async_copy(rows_0, acc.at[pl.ds(base, CHUNK)],
                                      gs_0).wait()
            pltpu.make_async_copy(rows_0.at[pl.ds(0, tail)],
                                  acc.at[pl.ds(base, tail)], gs_0).wait()

        @pl.when(sid < NS - 1)
        def _():
            zfill(ROWS_MAIN // CHUNK, ROWS_MAIN % CHUNK)

        @pl.when(sid == NS - 1)
        def _():
            zfill(ROWS_LAST // CHUNK, ROWS_LAST % CHUNK)

        plsc.subcore_barrier()


        def block_of(g):
            for bi, (start, cnt) in enumerate(BLOCKS):
                if start <= g < start + cnt:
                    return bi, start
            raise AssertionError(g)

        def load_idx(bi, sync):
            start, cnt = BLOCKS[bi]
            s, dd = sets[bi % 2]
            copies = ((src_hbm.at[wid, pl.ds(start, cnt)], s.at[pl.ds(0, cnt)]),
                      (dst_hbm.at[wid, pl.ds(start, cnt)], dd.at[pl.ds(0, cnt)]))
            for hsrc, vdst in copies:
                if sync:
                    pltpu.sync_copy(hsrc, vdst)
                else:
                    pltpu.async_copy(hsrc, vdst, isem)

        def wait_idx(bi):
            start, cnt = BLOCKS[bi]
            s, dd = sets[bi % 2]
            pltpu.make_async_copy(src_hbm.at[wid, pl.ds(start, cnt)],
                                  s.at[pl.ds(0, cnt)], isem).wait()
            pltpu.make_async_copy(dst_hbm.at[wid, pl.ds(start, cnt)],
                                  dd.at[pl.ds(0, cnt)], isem).wait()

        def issue_gather(g):
            bi, start = block_of(g)
            s = sets[bi % 2][0]
            pltpu.async_copy(x_hbm.at[s.at[g - start]], rows[g % 3],
                             gsem[g % 3])

        def chunk_static(g, wait_prev_scatter, gather_g):
            bi, start = block_of(g)
            s, dd = sets[bi % 2]
            b = g % 3
            pltpu.make_async_copy(x_hbm.at[s.at[g - start]], rows[b],
                                  gsem[b]).wait()
            pltpu.async_copy(rows[b], acc.at[dd.at[g - start]], ssem[b],
                             add=True)
            if gather_g is not None:
                if wait_prev_scatter:
                    nb = (b + 2) % 3
                    pltpu.make_async_copy(rows[nb], acc.at[dd.at[0]],
                                          ssem[nb]).wait()
                issue_gather(gather_g)

        load_idx(0, sync=True)
        load_idx(1, sync=False)
        issue_gather(0)
        issue_gather(1)

        for bi, (g0, cnt) in enumerate(BLOCKS):
            s, dd = sets[bi % 2]
            c0 = g0 % 3
            last = bi + 1 >= len(BLOCKS)

            chunk_static(g0, g0 >= 1, g0 + 2)
            chunk_static(g0 + 1, True, g0 + 3)
            if 1 <= bi < len(BLOCKS) - 1:
                load_idx(bi + 1, sync=False)

            def step(i, carry, s=s, dd=dd, c0=c0):
                for b in range(3):
                    nb = (b + 2) % 3

                    @pl.when(lax.rem(i + c0, 3) == b)
                    def _():
                        pltpu.make_async_copy(x_hbm.at[s.at[i]], rows[b],
                                              gsem[b]).wait()
                        pltpu.async_copy(rows[b], acc.at[dd.at[i]], ssem[b],
                                         add=True)
                        pltpu.make_async_copy(rows[nb], acc.at[dd.at[i]],
                                              ssem[nb]).wait()
                        pltpu.async_copy(x_hbm.at[s.at[i + 2]], rows[nb],
                                         gsem[nb])

                return carry

            lax.fori_loop(2, cnt - 2, step, 0)

            if not last:
                wait_idx(bi + 1)
                chunk_static(g0 + cnt - 2, True, g0 + cnt)
                chunk_static(g0 + cnt - 1, True, g0 + cnt + 1)
            else:
                chunk_static(g0 + cnt - 2, True, None)
                chunk_static(g0 + cnt - 1, False, None)

        for b in range(3):
            pltpu.make_async_copy(rows[b], acc.at[didx_a.at[0]],
                                  ssem[b]).wait()

        plsc.subcore_barrier()

        @pl.when(sid < NS - 1)
        def _():
            pltpu.sync_copy(acc.at[pl.ds(base, ROWS_MAIN)],
                            out_hbm.at[cid, pl.ds(base, ROWS_MAIN)])

        @pl.when(sid == NS - 1)
        def _():
            pltpu.sync_copy(acc.at[pl.ds(base, ROWS_LAST)],
                            out_hbm.at[cid, pl.ds(base, ROWS_LAST)])

    return body(src2d, dst2d, x)


def _dense_mid(parts, W1, b1):
    R = 400

    def body(a_ref, w1_ref, b1_ref, o_ref):
        a = a_ref[0] + a_ref[1]
        h = jnp.dot(a, w1_ref[...], preferred_element_type=jnp.float32)
        o_ref[...] = jnp.maximum(h + b1_ref[...], 0.0)

    return pl.pallas_call(
        body,
        grid=(N_NODES // R,),
        in_specs=[
            pl.BlockSpec((NC, R, D_IN), lambda i: (0, i, 0)),
            pl.BlockSpec((D_IN, D_IN), lambda i: (0, 0)),
            pl.BlockSpec((1, D_IN), lambda i: (0, 0)),
        ],
        out_specs=pl.BlockSpec((R, D_IN), lambda i: (i, 0)),
        out_shape=jax.ShapeDtypeStruct((N_NODES, D_IN), jnp.float32),
    )(parts, W1, b1.reshape(1, D_IN))


def _final_logsoftmax(parts2, W2, b2):
    R = 400

    def body(a_ref, w2_ref, b2_ref, o_ref):
        a = a_ref[0] + a_ref[1]
        v = jnp.dot(a, w2_ref[...], preferred_element_type=jnp.float32)
        v = v + b2_ref[...]
        m = jnp.max(v, axis=1, keepdims=True)
        e = jnp.exp(v - m)
        s = jnp.sum(e, axis=1, keepdims=True)
        o_ref[...] = (v - m) - jnp.log(s)

    return pl.pallas_call(
        body,
        grid=(N_NODES // R,),
        in_specs=[
            pl.BlockSpec((NC, R, D_IN), lambda i: (0, i, 0)),
            pl.BlockSpec((D_IN, D_OUT), lambda i: (0, 0)),
            pl.BlockSpec((1, D_OUT), lambda i: (0, 0)),
        ],
        out_specs=pl.BlockSpec((R, D_OUT), lambda i: (i, 0)),
        out_shape=jax.ShapeDtypeStruct((N_NODES, D_OUT), jnp.float32),
    )(parts2, W2, b2.reshape(1, D_OUT))


def kernel(x, edge_index, W1, b1, W2, b2):
    src = edge_index[0].astype(jnp.int32).reshape(NW, NCHUNK, CHUNK)
    dst = edge_index[1].astype(jnp.int32).reshape(NW, NCHUNK, CHUNK)
    parts1 = _seg_sum_partials(src, dst, x, D_IN)
    h = _dense_mid(parts1, W1, b1)
    parts2 = _seg_sum_partials(src, dst, h, D_IN)
    return _final_logsoftmax(parts2, W2, b2)


# speedup vs baseline: 3.9346x; 1.0114x over previous
import functools

import jax
import jax.numpy as jnp
from jax import lax
from jax.experimental import pallas as pl
from jax.experimental.pallas import tpu as pltpu
from jax.experimental.pallas import tpu_sc as plsc

N_NODES = 10000
N_EDGES = 320000
D_IN = 128
D_OUT = 16

NC = 2
NS = 16
NW = NC * NS
EPW = N_EDGES // NW
CHUNK = 80
NCHUNK = EPW // CHUNK
BLOCKS = ((0, 32), (32, 32), (64, 32), (96, NCHUNK - 96))
IBUF = 32
ROWS_MAIN = 632
ROWS_LAST = N_NODES - (NS - 1) * ROWS_MAIN


def _seg_sum_partials(src2d, dst2d, x, d):
    mesh = plsc.VectorSubcoreMesh(core_axis_name="c", subcore_axis_name="s")

    @functools.partial(
        pl.kernel,
        out_type=jax.ShapeDtypeStruct((NC, N_NODES, d), jnp.float32),
        mesh=mesh,
        scratch_types=[
            pltpu.VMEM((IBUF, CHUNK), jnp.int32),
            pltpu.VMEM((IBUF, CHUNK), jnp.int32),
            pltpu.VMEM((IBUF, CHUNK), jnp.int32),
            pltpu.VMEM((IBUF, CHUNK), jnp.int32),
            pltpu.VMEM((CHUNK, d), jnp.float32),
            pltpu.VMEM((CHUNK, d), jnp.float32),
            pltpu.VMEM((CHUNK, d), jnp.float32),
            pltpu.VMEM_SHARED((N_NODES, d), jnp.float32),
            pltpu.SemaphoreType.DMA,
            pltpu.SemaphoreType.DMA,
            pltpu.SemaphoreType.DMA,
            pltpu.SemaphoreType.DMA,
            pltpu.SemaphoreType.DMA,
            pltpu.SemaphoreType.DMA,
            pltpu.SemaphoreType.DMA,
        ],
    )
    def body(src_hbm, dst_hbm, x_hbm, out_hbm, sidx_a, didx_a, sidx_b, didx_b,
             rows_0, rows_1, rows_2, acc, gs_0, gs_1, gs_2, ss_0, ss_1, ss_2,
             isem):
        rows = (rows_0, rows_1, rows_2)
        gsem = (gs_0, gs_1, gs_2)
        ssem = (ss_0, ss_1, ss_2)
        sets = ((sidx_a, didx_a), (sidx_b, didx_b))
        cid = lax.axis_index("c")
        sid = lax.axis_index("s")
        wid = sid * NC + cid

        base = sid * ROWS_MAIN


        def block_of(g):
            for bi, (start, cnt) in enumerate(BLOCKS):
                if start <= g < start + cnt:
                    return bi, start
            raise AssertionError(g)

        def load_idx(bi, sync):
            start, cnt = BLOCKS[bi]
            s, dd = sets[bi % 2]
            copies = ((src_hbm.at[wid, pl.ds(start, cnt)], s.at[pl.ds(0, cnt)]),
                      (dst_hbm.at[wid, pl.ds(start, cnt)], dd.at[pl.ds(0, cnt)]))
            for hsrc, vdst in copies:
                if sync:
                    pltpu.sync_copy(hsrc, vdst)
                else:
                    pltpu.async_copy(hsrc, vdst, isem)

        def wait_idx(bi):
            start, cnt = BLOCKS[bi]
            s, dd = sets[bi % 2]
            pltpu.make_async_copy(src_hbm.at[wid, pl.ds(start, cnt)],
                                  s.at[pl.ds(0, cnt)], isem).wait()
            pltpu.make_async_copy(dst_hbm.at[wid, pl.ds(start, cnt)],
                                  dd.at[pl.ds(0, cnt)], isem).wait()

        def issue_gather(g):
            bi, start = block_of(g)
            s = sets[bi % 2][0]
            pltpu.async_copy(x_hbm.at[s.at[g - start]], rows[g % 3],
                             gsem[g % 3])

        def chunk_static(g, wait_prev_scatter, gather_g):
            bi, start = block_of(g)
            s, dd = sets[bi % 2]
            b = g % 3
            pltpu.make_async_copy(x_hbm.at[s.at[g - start]], rows[b],
                                  gsem[b]).wait()
            pltpu.async_copy(rows[b], acc.at[dd.at[g - start]], ssem[b],
                             add=True)
            if gather_g is not None:
                if wait_prev_scatter:
                    nb = (b + 2) % 3
                    pltpu.make_async_copy(rows[nb], acc.at[dd.at[0]],
                                          ssem[nb]).wait()
                issue_gather(gather_g)

        load_idx(0, sync=True)
        load_idx(1, sync=False)
        issue_gather(0)
        issue_gather(1)

        def zrow(i, carry):
            for j in range(d // 16):
                rows_2[i, pl.ds(j * 16, 16)] = jnp.zeros((16,), jnp.float32)
            return carry

        lax.fori_loop(0, CHUNK, zrow, 0)

        def zfill(nfull, tail):
            for k in range(nfull):
                pltpu.async_copy(rows_2, acc.at[pl.ds(base + k * CHUNK, CHUNK)],
                                 ss_0)
            pltpu.async_copy(rows_2.at[pl.ds(0, tail)],
                             acc.at[pl.ds(base + nfull * CHUNK, tail)], ss_0)
            for k in range(nfull):
                pltpu.make_async_copy(rows_2, acc.at[pl.ds(base, CHUNK)],
                                      ss_0).wait()
            pltpu.make_async_copy(rows_2.at[pl.ds(0, tail)],
                                  acc.at[pl.ds(base, tail)], ss_0).wait()

        @pl.when(sid < NS - 1)
        def _():
            zfill(ROWS_MAIN // CHUNK, ROWS_MAIN % CHUNK)

        @pl.when(sid == NS - 1)
        def _():
            zfill(ROWS_LAST // CHUNK, ROWS_LAST % CHUNK)

        plsc.subcore_barrier()

        for bi, (g0, cnt) in enumerate(BLOCKS):
            s, dd = sets[bi % 2]
            c0 = g0 % 3
            last = bi + 1 >= len(BLOCKS)

            chunk_static(g0, g0 >= 1, g0 + 2)
            chunk_static(g0 + 1, True, g0 + 3)
            if 1 <= bi < len(BLOCKS) - 1:
                load_idx(bi + 1, sync=False)

            def step(i, carry, s=s, dd=dd, c0=c0):
                for b in range(3):
                    nb = (b + 2) % 3

                    @pl.when(lax.rem(i + c0, 3) == b)
                    def _():
                        pltpu.make_async_copy(x_hbm.at[s.at[i]], rows[b],
                                              gsem[b]).wait()
                        pltpu.async_copy(rows[b], acc.at[dd.at[i]], ssem[b],
                                         add=True)
                        pltpu.make_async_copy(rows[nb], acc.at[dd.at[i]],
                                              ssem[nb]).wait()
                        pltpu.async_copy(x_hbm.at[s.at[i + 2]], rows[nb],
                                         gsem[nb])

                return carry

            lax.fori_loop(2, cnt - 2, step, 0)

            if not last:
                wait_idx(bi + 1)
                chunk_static(g0 + cnt - 2, True, g0 + cnt)
                chunk_static(g0 + cnt - 1, True, g0 + cnt + 1)
            else:
                chunk_static(g0 + cnt - 2, True, None)
                chunk_static(g0 + cnt - 1, False, None)

        for b in range(3):
            pltpu.make_async_copy(rows[b], acc.at[didx_a.at[0]],
                                  ssem[b]).wait()

        plsc.subcore_barrier()

        @pl.when(sid < NS - 1)
        def _():
            pltpu.sync_copy(acc.at[pl.ds(base, ROWS_MAIN)],
                            out_hbm.at[cid, pl.ds(base, ROWS_MAIN)])

        @pl.when(sid == NS - 1)
        def _():
            pltpu.sync_copy(acc.at[pl.ds(base, ROWS_LAST)],
                            out_hbm.at[cid, pl.ds(base, ROWS_LAST)])

    return body(src2d, dst2d, x)


def _dense_mid(parts, W1, b1):
    R = 400

    def body(a_ref, w1_ref, b1_ref, o_ref):
        a = a_ref[0] + a_ref[1]
        h = jnp.dot(a, w1_ref[...], preferred_element_type=jnp.float32)
        o_ref[...] = jnp.maximum(h + b1_ref[...], 0.0)

    return pl.pallas_call(
        body,
        grid=(N_NODES // R,),
        in_specs=[
            pl.BlockSpec((NC, R, D_IN), lambda i: (0, i, 0)),
            pl.BlockSpec((D_IN, D_IN), lambda i: (0, 0)),
            pl.BlockSpec((1, D_IN), lambda i: (0, 0)),
        ],
        out_specs=pl.BlockSpec((R, D_IN), lambda i: (i, 0)),
        out_shape=jax.ShapeDtypeStruct((N_NODES, D_IN), jnp.float32),
    )(parts, W1, b1.reshape(1, D_IN))


def _final_logsoftmax(parts2, W2, b2):
    R = 400

    def body(a_ref, w2_ref, b2_ref, o_ref):
        a = a_ref[0] + a_ref[1]
        v = jnp.dot(a, w2_ref[...], preferred_element_type=jnp.float32)
        v = v + b2_ref[...]
        m = jnp.max(v, axis=1, keepdims=True)
        e = jnp.exp(v - m)
        s = jnp.sum(e, axis=1, keepdims=True)
        o_ref[...] = (v - m) - jnp.log(s)

    return pl.pallas_call(
        body,
        grid=(N_NODES // R,),
        in_specs=[
            pl.BlockSpec((NC, R, D_IN), lambda i: (0, i, 0)),
            pl.BlockSpec((D_IN, D_OUT), lambda i: (0, 0)),
            pl.BlockSpec((1, D_OUT), lambda i: (0, 0)),
        ],
        out_specs=pl.BlockSpec((R, D_OUT), lambda i: (i, 0)),
        out_shape=jax.ShapeDtypeStruct((N_NODES, D_OUT), jnp.float32),
    )(parts2, W2, b2.reshape(1, D_OUT))


def kernel(x, edge_index, W1, b1, W2, b2):
    src = edge_index[0].astype(jnp.int32).reshape(NW, NCHUNK, CHUNK)
    dst = edge_index[1].astype(jnp.int32).reshape(NW, NCHUNK, CHUNK)
    parts1 = _seg_sum_partials(src, dst, x, D_IN)
    h = _dense_mid(parts1, W1, b1)
    parts2 = _seg_sum_partials(src, dst, h, D_IN)
    return _final_logsoftmax(parts2, W2, b2)


# speedup vs baseline: 4.2648x; 1.0839x over previous
import functools

import jax
import jax.numpy as jnp
from jax import lax
from jax.experimental import pallas as pl
from jax.experimental.pallas import tpu as pltpu
from jax.experimental.pallas import tpu_sc as plsc

N_NODES = 10000
N_EDGES = 320000
D_IN = 128
D_OUT = 16

NC = 2
NS = 16
NW = NC * NS
EPW = N_EDGES // NW
CHUNK = 80
NCHUNK = EPW // CHUNK
BLOCKS = ((0, 32), (32, 32), (64, 32), (96, NCHUNK - 96))
IBUF = 32
ROWS_MAIN = 632
ROWS_LAST = N_NODES - (NS - 1) * ROWS_MAIN


def _seg_sum_partials(src2d, dst2d, x, d):
    mesh = plsc.VectorSubcoreMesh(core_axis_name="c", subcore_axis_name="s")

    @functools.partial(
        pl.kernel,
        out_type=jax.ShapeDtypeStruct((NC, N_NODES, d), jnp.float32),
        mesh=mesh,
        scratch_types=[
            pltpu.VMEM((IBUF, CHUNK), jnp.int32),
            pltpu.VMEM((IBUF, CHUNK), jnp.int32),
            pltpu.VMEM((IBUF, CHUNK), jnp.int32),
            pltpu.VMEM((IBUF, CHUNK), jnp.int32),
            pltpu.VMEM((CHUNK, d), jnp.float32),
            pltpu.VMEM((CHUNK, d), jnp.float32),
            pltpu.VMEM((CHUNK, d), jnp.float32),
            pltpu.VMEM_SHARED((N_NODES, d), jnp.float32),
            pltpu.SemaphoreType.DMA,
            pltpu.SemaphoreType.DMA,
            pltpu.SemaphoreType.DMA,
            pltpu.SemaphoreType.DMA,
            pltpu.SemaphoreType.DMA,
            pltpu.SemaphoreType.DMA,
            pltpu.SemaphoreType.DMA,
        ],
    )
    def body(src_hbm, dst_hbm, x_hbm, out_hbm, sidx_a, didx_a, sidx_b, didx_b,
             rows_0, rows_1, rows_2, acc, gs_0, gs_1, gs_2, ss_0, ss_1, ss_2,
             isem):
        rows = (rows_0, rows_1, rows_2)
        gsem = (gs_0, gs_1, gs_2)
        ssem = (ss_0, ss_1, ss_2)
        sets = ((sidx_a, didx_a), (sidx_b, didx_b))
        cid = lax.axis_index("c")
        sid = lax.axis_index("s")
        wid = sid * NC + cid

        base = sid * ROWS_MAIN


        def block_of(g):
            for bi, (start, cnt) in enumerate(BLOCKS):
                if start <= g < start + cnt:
                    return bi, start
            raise AssertionError(g)

        def load_idx(bi, sync):
            start, cnt = BLOCKS[bi]
            s, dd = sets[bi % 2]
            copies = ((src_hbm.at[wid, pl.ds(start, cnt)], s.at[pl.ds(0, cnt)]),
                      (dst_hbm.at[wid, pl.ds(start, cnt)], dd.at[pl.ds(0, cnt)]))
            for hsrc, vdst in copies:
                if sync:
                    pltpu.sync_copy(hsrc, vdst)
                else:
                    pltpu.async_copy(hsrc, vdst, isem)

        def wait_idx(bi):
            start, cnt = BLOCKS[bi]
            s, dd = sets[bi % 2]
            pltpu.make_async_copy(src_hbm.at[wid, pl.ds(start, cnt)],
                                  s.at[pl.ds(0, cnt)], isem).wait()
            pltpu.make_async_copy(dst_hbm.at[wid, pl.ds(start, cnt)],
                                  dd.at[pl.ds(0, cnt)], isem).wait()

        def issue_gather(g):
            bi, start = block_of(g)
            s = sets[bi % 2][0]
            pltpu.async_copy(x_hbm.at[s.at[g - start]], rows[g % 3],
                             gsem[g % 3])

        def chunk_static(g, wait_prev_scatter, gather_g):
            bi, start = block_of(g)
            s, dd = sets[bi % 2]
            b = g % 3
            pltpu.make_async_copy(x_hbm.at[s.at[g - start]], rows[b],
                                  gsem[b]).wait()
            pltpu.async_copy(rows[b], acc.at[dd.at[g - start]], ssem[b],
                             add=True)
            if gather_g is not None:
                if wait_prev_scatter:
                    nb = (b + 2) % 3
                    pltpu.make_async_copy(rows[nb], acc.at[dd.at[0]],
                                          ssem[nb]).wait()
                issue_gather(gather_g)

        load_idx(0, sync=True)
        load_idx(1, sync=False)
        issue_gather(0)
        issue_gather(1)

        def zrow(i, carry):
            for j in range(d // 16):
                rows_2[i, pl.ds(j * 16, 16)] = jnp.zeros((16,), jnp.float32)
            return carry

        lax.fori_loop(0, CHUNK, zrow, 0)

        def zfill(nfull, tail):
            for k in range(nfull):
                pltpu.async_copy(rows_2, acc.at[pl.ds(base + k * CHUNK, CHUNK)],
                                 ss_0)
            pltpu.async_copy(rows_2.at[pl.ds(0, tail)],
                             acc.at[pl.ds(base + nfull * CHUNK, tail)], ss_0)
            for k in range(nfull):
                pltpu.make_async_copy(rows_2, acc.at[pl.ds(base, CHUNK)],
                                      ss_0).wait()
            pltpu.make_async_copy(rows_2.at[pl.ds(0, tail)],
                                  acc.at[pl.ds(base, tail)], ss_0).wait()

        @pl.when(sid < NS - 1)
        def _():
            zfill(ROWS_MAIN // CHUNK, ROWS_MAIN % CHUNK)

        @pl.when(sid == NS - 1)
        def _():
            zfill(ROWS_LAST // CHUNK, ROWS_LAST % CHUNK)

        plsc.subcore_barrier()

        for bi, (g0, cnt) in enumerate(BLOCKS):
            s, dd = sets[bi % 2]
            c0 = g0 % 3
            last = bi + 1 >= len(BLOCKS)

            chunk_static(g0, g0 >= 1, g0 + 2)
            chunk_static(g0 + 1, True, g0 + 3)
            if 1 <= bi < len(BLOCKS) - 1:
                load_idx(bi + 1, sync=False)

            def step(i, carry, s=s, dd=dd, c0=c0):
                for b in range(3):
                    nb = (b + 2) % 3

                    @pl.when(lax.rem(i + c0, 3) == b)
                    def _():
                        pltpu.make_async_copy(x_hbm.at[s.at[i]], rows[b],
                                              gsem[b]).wait()
                        pltpu.async_copy(rows[b], acc.at[dd.at[i]], ssem[b],
                                         add=True)
                        pltpu.make_async_copy(rows[nb], acc.at[dd.at[i]],
                                              ssem[nb]).wait()
                        pltpu.async_copy(x_hbm.at[s.at[i + 2]], rows[nb],
                                         gsem[nb])

                return carry

            lax.fori_loop(2, cnt - 2, step, 0)

            if not last:
                wait_idx(bi + 1)
                chunk_static(g0 + cnt - 2, True, g0 + cnt)
                chunk_static(g0 + cnt - 1, True, g0 + cnt + 1)
            else:
                chunk_static(g0 + cnt - 2, True, None)
                chunk_static(g0 + cnt - 1, False, None)

        for b in range(3):
            pltpu.make_async_copy(rows[b], acc.at[didx_a.at[0]],
                                  ssem[b]).wait()

        plsc.subcore_barrier()

        @pl.when(sid < NS - 1)
        def _():
            pltpu.sync_copy(acc.at[pl.ds(base, ROWS_MAIN)],
                            out_hbm.at[cid, pl.ds(base, ROWS_MAIN)])

        @pl.when(sid == NS - 1)
        def _():
            pltpu.sync_copy(acc.at[pl.ds(base, ROWS_LAST)],
                            out_hbm.at[cid, pl.ds(base, ROWS_LAST)])

    return body(src2d, dst2d, x)


def _dense_mid(parts, W1, b1):
    R = 2000

    def body(a_ref, w1_ref, b1_ref, o_ref):
        a = a_ref[0] + a_ref[1]
        h = jnp.dot(a, w1_ref[...], preferred_element_type=jnp.float32)
        o_ref[...] = jnp.maximum(h + b1_ref[...], 0.0)

    return pl.pallas_call(
        body,
        grid=(N_NODES // R,),
        in_specs=[
            pl.BlockSpec((NC, R, D_IN), lambda i: (0, i, 0)),
            pl.BlockSpec((D_IN, D_IN), lambda i: (0, 0)),
            pl.BlockSpec((1, D_IN), lambda i: (0, 0)),
        ],
        out_specs=pl.BlockSpec((R, D_IN), lambda i: (i, 0)),
        out_shape=jax.ShapeDtypeStruct((N_NODES, D_IN), jnp.float32),
    )(parts, W1, b1.reshape(1, D_IN))


def _final_logsoftmax(parts2, W2, b2):
    R = 2000

    def body(a_ref, w2_ref, b2_ref, o_ref):
        a = a_ref[0] + a_ref[1]
        v = jnp.dot(a, w2_ref[...], preferred_element_type=jnp.float32)
        v = v + b2_ref[...]
        m = jnp.max(v, axis=1, keepdims=True)
        e = jnp.exp(v - m)
        s = jnp.sum(e, axis=1, keepdims=True)
        o_ref[...] = (v - m) - jnp.log(s)

    return pl.pallas_call(
        body,
        grid=(N_NODES // R,),
        in_specs=[
            pl.BlockSpec((NC, R, D_IN), lambda i: (0, i, 0)),
            pl.BlockSpec((D_IN, D_OUT), lambda i: (0, 0)),
            pl.BlockSpec((1, D_OUT), lambda i: (0, 0)),
        ],
        out_specs=pl.BlockSpec((R, D_OUT), lambda i: (i, 0)),
        out_shape=jax.ShapeDtypeStruct((N_NODES, D_OUT), jnp.float32),
    )(parts2, W2, b2.reshape(1, D_OUT))


def kernel(x, edge_index, W1, b1, W2, b2):
    src = edge_index[0].astype(jnp.int32).reshape(NW, NCHUNK, CHUNK)
    dst = edge_index[1].astype(jnp.int32).reshape(NW, NCHUNK, CHUNK)
    parts1 = _seg_sum_partials(src, dst, x, D_IN)
    h = _dense_mid(parts1, W1, b1)
    parts2 = _seg_sum_partials(src, dst, h, D_IN)
    return _final_logsoftmax(parts2, W2, b2)


# speedup vs baseline: 4.2935x; 1.0067x over previous
import functools

import jax
import jax.numpy as jnp
from jax import lax
from jax.experimental import pallas as pl
from jax.experimental.pallas import tpu as pltpu
from jax.experimental.pallas import tpu_sc as plsc

N_NODES = 10000
N_EDGES = 320000
D_IN = 128
D_OUT = 16

NC = 2
NS = 16
NW = NC * NS
EPW = N_EDGES // NW
CHUNK = 80
NCHUNK = EPW // CHUNK
BLOCKS = ((0, 32), (32, 32), (64, 32), (96, NCHUNK - 96))
IBUF = 32
ROWS_MAIN = 632
ROWS_LAST = N_NODES - (NS - 1) * ROWS_MAIN


def _seg_sum_partials(src2d, dst2d, x, d):
    mesh = plsc.VectorSubcoreMesh(core_axis_name="c", subcore_axis_name="s")

    @functools.partial(
        pl.kernel,
        out_type=jax.ShapeDtypeStruct((NC, N_NODES, d), jnp.float32),
        mesh=mesh,
        scratch_types=[
            pltpu.VMEM((IBUF, CHUNK), jnp.int32),
            pltpu.VMEM((IBUF, CHUNK), jnp.int32),
            pltpu.VMEM((IBUF, CHUNK), jnp.int32),
            pltpu.VMEM((IBUF, CHUNK), jnp.int32),
            pltpu.VMEM((CHUNK, d), jnp.float32),
            pltpu.VMEM((CHUNK, d), jnp.float32),
            pltpu.VMEM((CHUNK, d), jnp.float32),
            pltpu.VMEM_SHARED((N_NODES, d), jnp.float32),
            pltpu.SemaphoreType.DMA,
            pltpu.SemaphoreType.DMA,
            pltpu.SemaphoreType.DMA,
            pltpu.SemaphoreType.DMA,
            pltpu.SemaphoreType.DMA,
            pltpu.SemaphoreType.DMA,
            pltpu.SemaphoreType.DMA,
        ],
    )
    def body(src_hbm, dst_hbm, x_hbm, out_hbm, sidx_a, didx_a, sidx_b, didx_b,
             rows_0, rows_1, rows_2, acc, gs_0, gs_1, gs_2, ss_0, ss_1, ss_2,
             isem):
        rows = (rows_0, rows_1, rows_2)
        gsem = (gs_0, gs_1, gs_2)
        ssem = (ss_0, ss_1, ss_2)
        sets = ((sidx_a, didx_a), (sidx_b, didx_b))
        cid = lax.axis_index("c")
        sid = lax.axis_index("s")
        wid = sid * NC + cid

        base = sid * ROWS_MAIN


        def block_of(g):
            for bi, (start, cnt) in enumerate(BLOCKS):
                if start <= g < start + cnt:
                    return bi, start
            raise AssertionError(g)

        def load_idx(bi, sync):
            start, cnt = BLOCKS[bi]
            s, dd = sets[bi % 2]
            copies = ((src_hbm.at[wid, pl.ds(start, cnt)], s.at[pl.ds(0, cnt)]),
                      (dst_hbm.at[wid, pl.ds(start, cnt)], dd.at[pl.ds(0, cnt)]))
            for hsrc, vdst in copies:
                if sync:
                    pltpu.sync_copy(hsrc, vdst)
                else:
                    pltpu.async_copy(hsrc, vdst, isem)

        def wait_idx(bi):
            start, cnt = BLOCKS[bi]
            s, dd = sets[bi % 2]
            pltpu.make_async_copy(src_hbm.at[wid, pl.ds(start, cnt)],
                                  s.at[pl.ds(0, cnt)], isem).wait()
            pltpu.make_async_copy(dst_hbm.at[wid, pl.ds(start, cnt)],
                                  dd.at[pl.ds(0, cnt)], isem).wait()

        def issue_gather(g):
            bi, start = block_of(g)
            s = sets[bi % 2][0]
            pltpu.async_copy(x_hbm.at[s.at[g - start]], rows[g % 3],
                             gsem[g % 3])

        def chunk_static(g, wait_prev_scatter, gather_g):
            bi, start = block_of(g)
            s, dd = sets[bi % 2]
            b = g % 3
            pltpu.make_async_copy(x_hbm.at[s.at[g - start]], rows[b],
                                  gsem[b]).wait()
            pltpu.async_copy(rows[b], acc.at[dd.at[g - start]], ssem[b],
                             add=True)
            if gather_g is not None:
                if wait_prev_scatter:
                    nb = (b + 2) % 3
                    pltpu.make_async_copy(rows[nb], acc.at[dd.at[0]],
                                          ssem[nb]).wait()
                issue_gather(gather_g)

        load_idx(0, sync=True)
        load_idx(1, sync=False)
        issue_gather(0)
        issue_gather(1)

        def zrow(i, carry):
            for j in range(d // 16):
                rows_2[i, pl.ds(j * 16, 16)] = jnp.zeros((16,), jnp.float32)
            return carry

        lax.fori_loop(0, CHUNK, zrow, 0)

        def zfill(nfull, tail):
            for k in range(nfull):
                pltpu.async_copy(rows_2, acc.at[pl.ds(base + k * CHUNK, CHUNK)],
                                 ss_0)
            pltpu.async_copy(rows_2.at[pl.ds(0, tail)],
                             acc.at[pl.ds(base + nfull * CHUNK, tail)], ss_0)
            for k in range(nfull):
                pltpu.make_async_copy(rows_2, acc.at[pl.ds(base, CHUNK)],
                                      ss_0).wait()
            pltpu.make_async_copy(rows_2.at[pl.ds(0, tail)],
                                  acc.at[pl.ds(base, tail)], ss_0).wait()

        @pl.when(sid < NS - 1)
        def _():
            zfill(ROWS_MAIN // CHUNK, ROWS_MAIN % CHUNK)

        @pl.when(sid == NS - 1)
        def _():
            zfill(ROWS_LAST // CHUNK, ROWS_LAST % CHUNK)

        plsc.subcore_barrier()

        for bi, (g0, cnt) in enumerate(BLOCKS):
            s, dd = sets[bi % 2]
            c0 = g0 % 3
            last = bi + 1 >= len(BLOCKS)

            chunk_static(g0, g0 >= 1, g0 + 2)
            chunk_static(g0 + 1, True, g0 + 3)
            if 1 <= bi < len(BLOCKS) - 1:
                load_idx(bi + 1, sync=False)

            def step(i, carry, s=s, dd=dd, c0=c0):
                for b in range(3):
                    nb = (b + 2) % 3

                    @pl.when(lax.rem(i + c0, 3) == b)
                    def _():
                        pltpu.make_async_copy(x_hbm.at[s.at[i]], rows[b],
                                              gsem[b]).wait()
                        pltpu.async_copy(rows[b], acc.at[dd.at[i]], ssem[b],
                                         add=True)
                        pltpu.make_async_copy(rows[nb], acc.at[dd.at[i]],
                                              ssem[nb]).wait()
                        pltpu.async_copy(x_hbm.at[s.at[i + 2]], rows[nb],
                                         gsem[nb])

                return carry

            lax.fori_loop(2, cnt - 2, step, 0)

            if not last:
                wait_idx(bi + 1)
                chunk_static(g0 + cnt - 2, True, g0 + cnt)
                chunk_static(g0 + cnt - 1, True, g0 + cnt + 1)
            else:
                chunk_static(g0 + cnt - 2, True, None)
                chunk_static(g0 + cnt - 1, False, None)

        for b in range(3):
            pltpu.make_async_copy(rows[b], acc.at[didx_a.at[0]],
                                  ssem[b]).wait()

        plsc.subcore_barrier()

        @pl.when(sid < NS - 1)
        def _():
            pltpu.sync_copy(acc.at[pl.ds(base, ROWS_MAIN)],
                            out_hbm.at[cid, pl.ds(base, ROWS_MAIN)])

        @pl.when(sid == NS - 1)
        def _():
            pltpu.sync_copy(acc.at[pl.ds(base, ROWS_LAST)],
                            out_hbm.at[cid, pl.ds(base, ROWS_LAST)])

    return body(src2d, dst2d, x)


def _dense_mid(parts, W1, b1):
    R = 10000

    def body(a_ref, w1_ref, b1_ref, o_ref):
        a = a_ref[0] + a_ref[1]
        h = jnp.dot(a, w1_ref[...], preferred_element_type=jnp.float32)
        o_ref[...] = jnp.maximum(h + b1_ref[...], 0.0)

    return pl.pallas_call(
        body,
        grid=(N_NODES // R,),
        in_specs=[
            pl.BlockSpec((NC, R, D_IN), lambda i: (0, i, 0)),
            pl.BlockSpec((D_IN, D_IN), lambda i: (0, 0)),
            pl.BlockSpec((1, D_IN), lambda i: (0, 0)),
        ],
        out_specs=pl.BlockSpec((R, D_IN), lambda i: (i, 0)),
        out_shape=jax.ShapeDtypeStruct((N_NODES, D_IN), jnp.float32),
    )(parts, W1, b1.reshape(1, D_IN))


def _final_logsoftmax(parts2, W2, b2):
    R = 10000

    def body(a_ref, w2_ref, b2_ref, o_ref):
        a = a_ref[0] + a_ref[1]
        v = jnp.dot(a, w2_ref[...], preferred_element_type=jnp.float32)
        v = v + b2_ref[...]
        m = jnp.max(v, axis=1, keepdims=True)
        e = jnp.exp(v - m)
        s = jnp.sum(e, axis=1, keepdims=True)
        o_ref[...] = (v - m) - jnp.log(s)

    return pl.pallas_call(
        body,
        grid=(N_NODES // R,),
        in_specs=[
            pl.BlockSpec((NC, R, D_IN), lambda i: (0, i, 0)),
            pl.BlockSpec((D_IN, D_OUT), lambda i: (0, 0)),
            pl.BlockSpec((1, D_OUT), lambda i: (0, 0)),
        ],
        out_specs=pl.BlockSpec((R, D_OUT), lambda i: (i, 0)),
        out_shape=jax.ShapeDtypeStruct((N_NODES, D_OUT), jnp.float32),
    )(parts2, W2, b2.reshape(1, D_OUT))


def kernel(x, edge_index, W1, b1, W2, b2):
    src = edge_index[0].astype(jnp.int32).reshape(NW, NCHUNK, CHUNK)
    dst = edge_index[1].astype(jnp.int32).reshape(NW, NCHUNK, CHUNK)
    parts1 = _seg_sum_partials(src, dst, x, D_IN)
    h = _dense_mid(parts1, W1, b1)
    parts2 = _seg_sum_partials(src, dst, h, D_IN)
    return _final_logsoftmax(parts2, W2, b2)


# speedup vs baseline: 4.4947x; 1.0469x over previous
import functools

import jax
import jax.numpy as jnp
from jax import lax
from jax.experimental import pallas as pl
from jax.experimental.pallas import tpu as pltpu
from jax.experimental.pallas import tpu_sc as plsc

N_NODES = 10000
N_EDGES = 320000
D_IN = 128
D_OUT = 16

NC = 2
NS = 16
NW = NC * NS
EPW = N_EDGES // NW
CHUNK = 80
NCHUNK = EPW // CHUNK
BLOCKS = ((0, 32), (32, 32), (64, 32), (96, NCHUNK - 96))
IBUF = 32
ROWS_MAIN = 632
ROWS_LAST = N_NODES - (NS - 1) * ROWS_MAIN


def _seg_sum_partials(ei, x, d):
    mesh = plsc.VectorSubcoreMesh(core_axis_name="c", subcore_axis_name="s")

    @functools.partial(
        pl.kernel,
        out_type=jax.ShapeDtypeStruct((NC, N_NODES, d), jnp.float32),
        mesh=mesh,
        scratch_types=[
            pltpu.VMEM((IBUF, CHUNK), jnp.int32),
            pltpu.VMEM((IBUF, CHUNK), jnp.int32),
            pltpu.VMEM((IBUF, CHUNK), jnp.int32),
            pltpu.VMEM((IBUF, CHUNK), jnp.int32),
            pltpu.VMEM((CHUNK, d), jnp.float32),
            pltpu.VMEM((CHUNK, d), jnp.float32),
            pltpu.VMEM((CHUNK, d), jnp.float32),
            pltpu.VMEM_SHARED((N_NODES, d), jnp.float32),
            pltpu.SemaphoreType.DMA,
            pltpu.SemaphoreType.DMA,
            pltpu.SemaphoreType.DMA,
            pltpu.SemaphoreType.DMA,
            pltpu.SemaphoreType.DMA,
            pltpu.SemaphoreType.DMA,
            pltpu.SemaphoreType.DMA,
        ],
    )
    def body(ei_hbm, x_hbm, out_hbm, sidx_a, didx_a, sidx_b, didx_b,
             rows_0, rows_1, rows_2, acc, gs_0, gs_1, gs_2, ss_0, ss_1, ss_2,
             isem):
        rows = (rows_0, rows_1, rows_2)
        gsem = (gs_0, gs_1, gs_2)
        ssem = (ss_0, ss_1, ss_2)
        sets = ((sidx_a, didx_a), (sidx_b, didx_b))
        cid = lax.axis_index("c")
        sid = lax.axis_index("s")
        wid = sid * NC + cid

        base = sid * ROWS_MAIN


        def block_of(g):
            for bi, (start, cnt) in enumerate(BLOCKS):
                if start <= g < start + cnt:
                    return bi, start
            raise AssertionError(g)

        def load_idx(bi, sync):
            start, cnt = BLOCKS[bi]
            s, dd = sets[bi % 2]
            copies = ((ei_hbm.at[0, wid, pl.ds(start, cnt)], s.at[pl.ds(0, cnt)]),
                      (ei_hbm.at[1, wid, pl.ds(start, cnt)], dd.at[pl.ds(0, cnt)]))
            for hsrc, vdst in copies:
                if sync:
                    pltpu.sync_copy(hsrc, vdst)
                else:
                    pltpu.async_copy(hsrc, vdst, isem)

        def wait_idx(bi):
            start, cnt = BLOCKS[bi]
            s, dd = sets[bi % 2]
            pltpu.make_async_copy(ei_hbm.at[0, wid, pl.ds(start, cnt)],
                                  s.at[pl.ds(0, cnt)], isem).wait()
            pltpu.make_async_copy(ei_hbm.at[1, wid, pl.ds(start, cnt)],
                                  dd.at[pl.ds(0, cnt)], isem).wait()

        def issue_gather(g):
            bi, start = block_of(g)
            s = sets[bi % 2][0]
            pltpu.async_copy(x_hbm.at[s.at[g - start]], rows[g % 3],
                             gsem[g % 3])

        def chunk_static(g, wait_prev_scatter, gather_g):
            bi, start = block_of(g)
            s, dd = sets[bi % 2]
            b = g % 3
            pltpu.make_async_copy(x_hbm.at[s.at[g - start]], rows[b],
                                  gsem[b]).wait()
            pltpu.async_copy(rows[b], acc.at[dd.at[g - start]], ssem[b],
                             add=True)
            if gather_g is not None:
                if wait_prev_scatter:
                    nb = (b + 2) % 3
                    pltpu.make_async_copy(rows[nb], acc.at[dd.at[0]],
                                          ssem[nb]).wait()
                issue_gather(gather_g)

        load_idx(0, sync=True)
        load_idx(1, sync=False)
        issue_gather(0)
        issue_gather(1)

        def zrow(i, carry):
            for j in range(d // 16):
                rows_2[i, pl.ds(j * 16, 16)] = jnp.zeros((16,), jnp.float32)
            return carry

        lax.fori_loop(0, CHUNK, zrow, 0)

        def zfill(nfull, tail):
            for k in range(nfull):
                pltpu.async_copy(rows_2, acc.at[pl.ds(base + k * CHUNK, CHUNK)],
                                 ss_0)
            pltpu.async_copy(rows_2.at[pl.ds(0, tail)],
                             acc.at[pl.ds(base + nfull * CHUNK, tail)], ss_0)
            for k in range(nfull):
                pltpu.make_async_copy(rows_2, acc.at[pl.ds(base, CHUNK)],
                                      ss_0).wait()
            pltpu.make_async_copy(rows_2.at[pl.ds(0, tail)],
                                  acc.at[pl.ds(base, tail)], ss_0).wait()

        @pl.when(sid < NS - 1)
        def _():
            zfill(ROWS_MAIN // CHUNK, ROWS_MAIN % CHUNK)

        @pl.when(sid == NS - 1)
        def _():
            zfill(ROWS_LAST // CHUNK, ROWS_LAST % CHUNK)

        plsc.subcore_barrier()

        for bi, (g0, cnt) in enumerate(BLOCKS):
            s, dd = sets[bi % 2]
            c0 = g0 % 3
            last = bi + 1 >= len(BLOCKS)

            chunk_static(g0, g0 >= 1, g0 + 2)
            chunk_static(g0 + 1, True, g0 + 3)
            if 1 <= bi < len(BLOCKS) - 1:
                load_idx(bi + 1, sync=False)

            def step(i, carry, s=s, dd=dd, c0=c0):
                for b in range(3):
                    nb = (b + 2) % 3

                    @pl.when(lax.rem(i + c0, 3) == b)
                    def _():
                        pltpu.make_async_copy(x_hbm.at[s.at[i]], rows[b],
                                              gsem[b]).wait()
                        pltpu.async_copy(rows[b], acc.at[dd.at[i]], ssem[b],
                                         add=True)
                        pltpu.make_async_copy(rows[nb], acc.at[dd.at[i]],
                                              ssem[nb]).wait()
                        pltpu.async_copy(x_hbm.at[s.at[i + 2]], rows[nb],
                                         gsem[nb])

                return carry

            lax.fori_loop(2, cnt - 2, step, 0)

            if not last:
                wait_idx(bi + 1)
                chunk_static(g0 + cnt - 2, True, g0 + cnt)
                chunk_static(g0 + cnt - 1, True, g0 + cnt + 1)
            else:
                chunk_static(g0 + cnt - 2, True, None)
                chunk_static(g0 + cnt - 1, False, None)

        for b in range(3):
            pltpu.make_async_copy(rows[b], acc.at[didx_a.at[0]],
                                  ssem[b]).wait()

        plsc.subcore_barrier()

        @pl.when(sid < NS - 1)
        def _():
            pltpu.sync_copy(acc.at[pl.ds(base, ROWS_MAIN)],
                            out_hbm.at[cid, pl.ds(base, ROWS_MAIN)])

        @pl.when(sid == NS - 1)
        def _():
            pltpu.sync_copy(acc.at[pl.ds(base, ROWS_LAST)],
                            out_hbm.at[cid, pl.ds(base, ROWS_LAST)])

    return body(ei, x)


def _dense_mid(parts, W1, b1):
    R = 10000

    def body(a_ref, w1_ref, b1_ref, o_ref):
        a = a_ref[0] + a_ref[1]
        h = jnp.dot(a, w1_ref[...], preferred_element_type=jnp.float32)
        o_ref[...] = jnp.maximum(h + b1_ref[...], 0.0)

    return pl.pallas_call(
        body,
        grid=(N_NODES // R,),
        in_specs=[
            pl.BlockSpec((NC, R, D_IN), lambda i: (0, i, 0)),
            pl.BlockSpec((D_IN, D_IN), lambda i: (0, 0)),
            pl.BlockSpec((1, D_IN), lambda i: (0, 0)),
        ],
        out_specs=pl.BlockSpec((R, D_IN), lambda i: (i, 0)),
        out_shape=jax.ShapeDtypeStruct((N_NODES, D_IN), jnp.float32),
    )(parts, W1, b1.reshape(1, D_IN))


def _final_logsoftmax(parts2, W2, b2):
    R = 10000

    def body(a_ref, w2_ref, b2_ref, o_ref):
        a = a_ref[0] + a_ref[1]
        v = jnp.dot(a, w2_ref[...], preferred_element_type=jnp.float32)
        v = v + b2_ref[...]
        m = jnp.max(v, axis=1, keepdims=True)
        e = jnp.exp(v - m)
        s = jnp.sum(e, axis=1, keepdims=True)
        o_ref[...] = (v - m) - jnp.log(s)

    return pl.pallas_call(
        body,
        grid=(N_NODES // R,),
        in_specs=[
            pl.BlockSpec((NC, R, D_IN), lambda i: (0, i, 0)),
            pl.BlockSpec((D_IN, D_OUT), lambda i: (0, 0)),
            pl.BlockSpec((1, D_OUT), lambda i: (0, 0)),
        ],
        out_specs=pl.BlockSpec((R, D_OUT), lambda i: (i, 0)),
        out_shape=jax.ShapeDtypeStruct((N_NODES, D_OUT), jnp.float32),
    )(parts2, W2, b2.reshape(1, D_OUT))


def kernel(x, edge_index, W1, b1, W2, b2):
    ei = edge_index.astype(jnp.int32).reshape(2, NW, NCHUNK, CHUNK)
    parts1 = _seg_sum_partials(ei, x, D_IN)
    h = _dense_mid(parts1, W1, b1)
    parts2 = _seg_sum_partials(ei, h, D_IN)
    return _final_logsoftmax(parts2, W2, b2)
